# Initial kernel scaffold; baseline (speedup 1.0000x reference)
import jax
import jax.numpy as jnp
from jax.experimental import pallas as pl


def kernel(x, edge_index, W1l, W1r, b1, W2l, W2r, b2, W3l, W3r, b3, Wlin, blin):
    raise NotImplementedError("write your pallas kernel here")


# speedup vs baseline: 8.3291x; 8.3291x over previous
import functools

import jax
import jax.numpy as jnp
from jax import lax
from jax.experimental import pallas as pl
from jax.experimental.pallas import tpu as pltpu
from jax.experimental.pallas import tpu_sc as plsc

N = 10000
E = 320000
D = 128
C = 64

NC = 2
NS = 16
NW = NC * NS

CHUNK = 128
NCH = 80
EPW = NCH * CHUNK
E_PAD = EPW * NW
N_ACC = 10112
ROWS_PER_TILE = N_ACC // NS
N_CNT = 16384
CNT_PER_TILE = N_CNT // NS

_f32 = jnp.float32


def _make_sc_agg(with_counts: bool):
  out_type = [jax.ShapeDtypeStruct((NC, N_ACC, D), _f32)]
  if with_counts:
    out_type.append(jax.ShapeDtypeStruct((NC * N_CNT,), _f32))

  scratch = [
      pltpu.VMEM_SHARED((N_ACC, D), _f32),
      pltpu.VMEM((NCH, CHUNK), jnp.int32),
      pltpu.VMEM((NCH, CHUNK), jnp.int32),
      pltpu.VMEM((CHUNK, D), _f32),
      pltpu.SemaphoreType.DMA,
  ]
  if with_counts:
    scratch.insert(1, pltpu.VMEM_SHARED((N_CNT,), _f32))
    scratch.append(pltpu.VMEM((CHUNK,), _f32))

  def body(*refs):
    if with_counts:
      (x_hbm, src_hbm, dst_hbm, z_hbm, zc_hbm, p_hbm, cnt_hbm,
       acc_sh, cnt_sh, sidx, didx, rows, sem, ones_v) = refs
    else:
      (x_hbm, src_hbm, dst_hbm, z_hbm, p_hbm,
       acc_sh, sidx, didx, rows, sem) = refs

    c = lax.axis_index("c")
    s = lax.axis_index("s")
    w = c * NS + s

    pltpu.sync_copy(src_hbm.at[pl.ds(w * NCH, NCH)], sidx)
    pltpu.sync_copy(dst_hbm.at[pl.ds(w * NCH, NCH)], didx)

    pltpu.sync_copy(z_hbm.at[pl.ds(s * ROWS_PER_TILE, ROWS_PER_TILE)],
                    acc_sh.at[pl.ds(s * ROWS_PER_TILE, ROWS_PER_TILE)])
    if with_counts:
      pltpu.sync_copy(zc_hbm.at[pl.ds(s * CNT_PER_TILE, CNT_PER_TILE)],
                      cnt_sh.at[pl.ds(s * CNT_PER_TILE, CNT_PER_TILE)])
      for k in range(CHUNK // 16):
        ones_v[pl.ds(16 * k, 16)] = jnp.ones((16,), _f32)
    plsc.subcore_barrier()

    def step(j, carry):
      pltpu.async_copy(x_hbm.at[sidx.at[j]], rows, sem).wait()
      pltpu.sync_copy(rows, acc_sh.at[didx.at[j]], add=True)
      if with_counts:
        pltpu.sync_copy(ones_v, cnt_sh.at[didx.at[j]], add=True)
      return carry

    lax.fori_loop(0, NCH, step, 0)
    plsc.subcore_barrier()

    pltpu.sync_copy(acc_sh.at[pl.ds(s * ROWS_PER_TILE, ROWS_PER_TILE)],
                    p_hbm.at[c].at[pl.ds(s * ROWS_PER_TILE, ROWS_PER_TILE)])
    if with_counts:
      pltpu.sync_copy(
          cnt_sh.at[pl.ds(s * CNT_PER_TILE, CNT_PER_TILE)],
          cnt_hbm.at[pl.ds(c * N_CNT + s * CNT_PER_TILE, CNT_PER_TILE)])

  return pl.kernel(
      body,
      out_type=tuple(out_type) if with_counts else out_type[0],
      mesh=plsc.VectorSubcoreMesh(core_axis_name="c", subcore_axis_name="s"),
      scratch_types=scratch,
  )


_sc_agg_counts = _make_sc_agg(True)
_sc_agg = _make_sc_agg(False)

_GRID = 5
_R = N // _GRID


def _dense_body(p0, p1, c0, c1, x, wl, wr, b, o):
  cnt = jnp.maximum(c0[...] + c1[...], 1.0)
  mean = (p0[...] + p1[...]) / cnt
  h = (jnp.dot(mean, wl[...], preferred_element_type=_f32)
       + jnp.dot(x[...], wr[...], preferred_element_type=_f32) + b[...])
  o[...] = jnp.maximum(h, 0.0)


def _dense_final_body(p0, p1, c0, c1, x, wl, wr, b, wlin, blin, o):
  cnt = jnp.maximum(c0[...] + c1[...], 1.0)
  mean = (p0[...] + p1[...]) / cnt
  h = (jnp.dot(mean, wl[...], preferred_element_type=_f32)
       + jnp.dot(x[...], wr[...], preferred_element_type=_f32) + b[...])
  h = jnp.maximum(h, 0.0)
  o[...] = jnp.dot(h, wlin[...], preferred_element_type=_f32) + blin[...]


def _row_spec(d):
  return pl.BlockSpec((_R, d), lambda i: (i, 0))


def _full_spec(r, d):
  return pl.BlockSpec((r, d), lambda i: (0, 0))


_dense = pl.pallas_call(
    _dense_body,
    grid=(_GRID,),
    in_specs=[_row_spec(D), _row_spec(D), _row_spec(1), _row_spec(1),
              _row_spec(D), _full_spec(D, D), _full_spec(D, D),
              _full_spec(1, D)],
    out_specs=_row_spec(D),
    out_shape=jax.ShapeDtypeStruct((N, D), _f32),
)

_dense_final = pl.pallas_call(
    _dense_final_body,
    grid=(_GRID,),
    in_specs=[_row_spec(D), _row_spec(D), _row_spec(1), _row_spec(1),
              _row_spec(D), _full_spec(D, D), _full_spec(D, D),
              _full_spec(1, D), _full_spec(D, C), _full_spec(1, C)],
    out_specs=pl.BlockSpec((_R, C), lambda i: (i, 0)),
    out_shape=jax.ShapeDtypeStruct((N, C), _f32),
)


def kernel(x, edge_index, W1l, W1r, b1, W2l, W2r, b2, W3l, W3r, b3,
           Wlin, blin):
  src = edge_index[0].astype(jnp.int32)
  dst = edge_index[1].astype(jnp.int32)
  npad = E_PAD - E
  pad_iota = jnp.arange(npad, dtype=jnp.int32)
  src_p = jnp.concatenate([src, pad_iota % 64]).reshape(E_PAD // CHUNK, CHUNK)
  dst_p = jnp.concatenate([dst, N + (pad_iota % 16)]).reshape(
      E_PAD // CHUNK, CHUNK)

  zrows = jnp.zeros((N_ACC, D), _f32)
  zcnt = jnp.zeros((N_CNT,), _f32)

  p, cnt = _sc_agg_counts(x, src_p, dst_p, zrows, zcnt)
  cnt = cnt.reshape(NC, N_CNT)
  c0 = cnt[0, :N, None]
  c1 = cnt[1, :N, None]
  b1_ = b1.reshape(1, D)
  h = _dense(p[0, :N], p[1, :N], c0, c1, x, W1l, W1r, b1_)

  p = _sc_agg(h, src_p, dst_p, zrows)
  h = _dense(p[0, :N], p[1, :N], c0, c1, h, W2l, W2r, b2.reshape(1, D))

  p = _sc_agg(h, src_p, dst_p, zrows)
  out = _dense_final(p[0, :N], p[1, :N], c0, c1, h, W3l, W3r,
                     b3.reshape(1, D), Wlin, blin.reshape(1, C))
  return out


# speedup vs baseline: 10.6551x; 1.2793x over previous
import functools

import jax
import jax.numpy as jnp
from jax import lax
from jax.experimental import pallas as pl
from jax.experimental.pallas import tpu as pltpu
from jax.experimental.pallas import tpu_sc as plsc

N = 10000
E = 320000
D = 128
C = 64

NC = 2
NS = 16
NW = NC * NS

CHUNK = 128
NCH = 80
SEG = 40
EPW = NCH * CHUNK
E_PAD = EPW * NW
N_ACC = 10112
ROWS_PER_TILE = N_ACC // NS
N_CNT = 16384
CNT_PER_TILE = N_CNT // NS

_f32 = jnp.float32


def _make_sc_agg(with_counts: bool):
  out_type = [jax.ShapeDtypeStruct((NC, N_ACC, D), _f32)]
  if with_counts:
    out_type.append(jax.ShapeDtypeStruct((NC * N_CNT,), _f32))

  scratch = [
      pltpu.VMEM_SHARED((N_ACC, D), _f32),
      pltpu.VMEM((SEG, CHUNK), jnp.int32),
      pltpu.VMEM((SEG, CHUNK), jnp.int32),
      pltpu.VMEM((CHUNK, D), _f32),
      pltpu.VMEM((CHUNK, D), _f32),
      pltpu.SemaphoreType.DMA,
      pltpu.SemaphoreType.DMA,
  ]
  if with_counts:
    scratch.insert(1, pltpu.VMEM_SHARED((N_CNT,), _f32))
    scratch.append(pltpu.VMEM((CHUNK,), _f32))
    scratch.append(pltpu.SemaphoreType.DMA)

  def body(*refs):
    if with_counts:
      (x_hbm, src_hbm, dst_hbm, z_hbm, zc_hbm, p_hbm, cnt_hbm,
       acc_sh, cnt_sh, sidx, didx, rows0, rows1, gsem, ssem,
       ones_v, csem) = refs
    else:
      (x_hbm, src_hbm, dst_hbm, z_hbm, p_hbm,
       acc_sh, sidx, didx, rows0, rows1, gsem, ssem) = refs

    c = lax.axis_index("c")
    s = lax.axis_index("s")
    w = c * NS + s

    pltpu.sync_copy(z_hbm.at[pl.ds(s * ROWS_PER_TILE, ROWS_PER_TILE)],
                    acc_sh.at[pl.ds(s * ROWS_PER_TILE, ROWS_PER_TILE)])
    if with_counts:
      pltpu.sync_copy(zc_hbm.at[pl.ds(s * CNT_PER_TILE, CNT_PER_TILE)],
                      cnt_sh.at[pl.ds(s * CNT_PER_TILE, CNT_PER_TILE)])
      for k in range(CHUNK // 16):
        ones_v[pl.ds(16 * k, 16)] = jnp.ones((16,), _f32)
    plsc.subcore_barrier()

    def start_gather(j, buf):
      pltpu.async_copy(x_hbm.at[sidx.at[j]], buf, gsem)

    def wait_gather(j, buf):
      pltpu.make_async_copy(x_hbm.at[sidx.at[j]], buf, gsem).wait()

    def start_scat(j, buf):
      pltpu.async_copy(buf, acc_sh.at[didx.at[j]], ssem, add=True)
      if with_counts:
        pltpu.async_copy(ones_v, cnt_sh.at[didx.at[j]], csem, add=True)

    def wait_scat(j, buf):
      pltpu.make_async_copy(buf, acc_sh.at[didx.at[j]], ssem).wait()
      if with_counts:
        pltpu.make_async_copy(ones_v, cnt_sh.at[didx.at[j]], csem).wait()

    for h in range(NCH // SEG):
      pltpu.sync_copy(src_hbm.at[pl.ds(w * NCH + h * SEG, SEG)], sidx)
      pltpu.sync_copy(dst_hbm.at[pl.ds(w * NCH + h * SEG, SEG)], didx)

      start_gather(0, rows0)
      wait_gather(0, rows0)
      start_gather(1, rows1)
      start_scat(0, rows0)

      def pair(t, carry):
        for b, (cur, oth) in ((0, (rows1, rows0)), (1, (rows0, rows1))):
          j = 2 * t + 1 + b
          wait_gather(j, cur)
          wait_scat(j - 1, oth)
          start_gather(j + 1, oth)
          start_scat(j, cur)
        return carry

      lax.fori_loop(0, (SEG - 2) // 2, pair, 0)

      j = SEG - 1
      wait_gather(j, rows1)
      wait_scat(j - 1, rows0)
      start_scat(j, rows1)
      wait_scat(j, rows1)
    plsc.subcore_barrier()

    pltpu.sync_copy(acc_sh.at[pl.ds(s * ROWS_PER_TILE, ROWS_PER_TILE)],
                    p_hbm.at[c].at[pl.ds(s * ROWS_PER_TILE, ROWS_PER_TILE)])
    if with_counts:
      pltpu.sync_copy(
          cnt_sh.at[pl.ds(s * CNT_PER_TILE, CNT_PER_TILE)],
          cnt_hbm.at[pl.ds(c * N_CNT + s * CNT_PER_TILE, CNT_PER_TILE)])

  return pl.kernel(
      body,
      out_type=tuple(out_type) if with_counts else out_type[0],
      mesh=plsc.VectorSubcoreMesh(core_axis_name="c", subcore_axis_name="s"),
      scratch_types=scratch,
  )


_sc_agg_counts = _make_sc_agg(True)
_sc_agg = _make_sc_agg(False)

_GRID = 5
_R = N // _GRID


def _dense_body(p0, p1, c0, c1, x, wl, wr, b, o):
  cnt = jnp.maximum(c0[...] + c1[...], 1.0)
  mean = (p0[...] + p1[...]) / cnt
  h = (jnp.dot(mean, wl[...], preferred_element_type=_f32)
       + jnp.dot(x[...], wr[...], preferred_element_type=_f32) + b[...])
  o[...] = jnp.maximum(h, 0.0)


def _dense_final_body(p0, p1, c0, c1, x, wl, wr, b, wlin, blin, o):
  cnt = jnp.maximum(c0[...] + c1[...], 1.0)
  mean = (p0[...] + p1[...]) / cnt
  h = (jnp.dot(mean, wl[...], preferred_element_type=_f32)
       + jnp.dot(x[...], wr[...], preferred_element_type=_f32) + b[...])
  h = jnp.maximum(h, 0.0)
  o[...] = jnp.dot(h, wlin[...], preferred_element_type=_f32) + blin[...]


def _row_spec(d):
  return pl.BlockSpec((_R, d), lambda i: (i, 0))


def _full_spec(r, d):
  return pl.BlockSpec((r, d), lambda i: (0, 0))


_dense = pl.pallas_call(
    _dense_body,
    grid=(_GRID,),
    in_specs=[_row_spec(D), _row_spec(D), _row_spec(1), _row_spec(1),
              _row_spec(D), _full_spec(D, D), _full_spec(D, D),
              _full_spec(1, D)],
    out_specs=_row_spec(D),
    out_shape=jax.ShapeDtypeStruct((N, D), _f32),
)

_dense_final = pl.pallas_call(
    _dense_final_body,
    grid=(_GRID,),
    in_specs=[_row_spec(D), _row_spec(D), _row_spec(1), _row_spec(1),
              _row_spec(D), _full_spec(D, D), _full_spec(D, D),
              _full_spec(1, D), _full_spec(D, C), _full_spec(1, C)],
    out_specs=pl.BlockSpec((_R, C), lambda i: (i, 0)),
    out_shape=jax.ShapeDtypeStruct((N, C), _f32),
)


def kernel(x, edge_index, W1l, W1r, b1, W2l, W2r, b2, W3l, W3r, b3,
           Wlin, blin):
  src = edge_index[0].astype(jnp.int32)
  dst = edge_index[1].astype(jnp.int32)
  npad = E_PAD - E
  pad_iota = jnp.arange(npad, dtype=jnp.int32)
  src_p = jnp.concatenate([src, pad_iota % 64]).reshape(E_PAD // CHUNK, CHUNK)
  dst_p = jnp.concatenate([dst, N + (pad_iota % 16)]).reshape(
      E_PAD // CHUNK, CHUNK)

  zrows = jnp.zeros((N_ACC, D), _f32)
  zcnt = jnp.zeros((N_CNT,), _f32)

  p, cnt = _sc_agg_counts(x, src_p, dst_p, zrows, zcnt)
  cnt = cnt.reshape(NC, N_CNT)
  c0 = cnt[0, :N, None]
  c1 = cnt[1, :N, None]
  b1_ = b1.reshape(1, D)
  h = _dense(p[0, :N], p[1, :N], c0, c1, x, W1l, W1r, b1_)

  p = _sc_agg(h, src_p, dst_p, zrows)
  h = _dense(p[0, :N], p[1, :N], c0, c1, h, W2l, W2r, b2.reshape(1, D))

  p = _sc_agg(h, src_p, dst_p, zrows)
  out = _dense_final(p[0, :N], p[1, :N], c0, c1, h, W3l, W3r,
                     b3.reshape(1, D), Wlin, blin.reshape(1, C))
  return out


# speedup vs baseline: 10.8908x; 1.0221x over previous
import functools

import jax
import jax.numpy as jnp
from jax import lax
from jax.experimental import pallas as pl
from jax.experimental.pallas import tpu as pltpu
from jax.experimental.pallas import tpu_sc as plsc

N = 10000
E = 320000
D = 128
C = 64

NC = 2
NS = 16
NW = NC * NS

CHUNK = 128
NCH = 80
SEG = 40
EPW = NCH * CHUNK
E_PAD = EPW * NW
N_ACC = 10112
ROWS_PER_TILE = N_ACC // NS
N_CNT = 16384
CNT_PER_TILE = N_CNT // NS

_f32 = jnp.float32


def _make_sc_agg(with_counts: bool):
  out_type = [jax.ShapeDtypeStruct((NC, N_ACC, D), _f32)]
  if with_counts:
    out_type.append(jax.ShapeDtypeStruct((NC * N_CNT,), _f32))

  scratch = [
      pltpu.VMEM_SHARED((N_ACC, D), _f32),
      pltpu.VMEM((SEG, CHUNK), jnp.int32),
      pltpu.VMEM((SEG, CHUNK), jnp.int32),
      pltpu.VMEM((CHUNK, D), _f32),
      pltpu.VMEM((CHUNK, D), _f32),
      pltpu.SemaphoreType.DMA,
      pltpu.SemaphoreType.DMA,
      pltpu.SemaphoreType.DMA,
  ]
  if with_counts:
    scratch.insert(1, pltpu.VMEM_SHARED((N_CNT,), _f32))
    scratch.append(pltpu.VMEM((CHUNK,), _f32))
    scratch.append(pltpu.SemaphoreType.DMA)

  def body(*refs):
    if with_counts:
      (x_hbm, src_hbm, dst_hbm, z_hbm, zc_hbm, p_hbm, cnt_hbm,
       acc_sh, cnt_sh, sidx, didx, rows0, rows1, gsem, ssem, zsem,
       ones_v, csem) = refs
    else:
      (x_hbm, src_hbm, dst_hbm, z_hbm, p_hbm,
       acc_sh, sidx, didx, rows0, rows1, gsem, ssem, zsem) = refs

    c = lax.axis_index("c")
    s = lax.axis_index("s")
    w = c * NS + s

    pltpu.async_copy(z_hbm.at[pl.ds(s * ROWS_PER_TILE, ROWS_PER_TILE)],
                     acc_sh.at[pl.ds(s * ROWS_PER_TILE, ROWS_PER_TILE)],
                     zsem)
    if with_counts:
      pltpu.async_copy(zc_hbm.at[pl.ds(s * CNT_PER_TILE, CNT_PER_TILE)],
                       cnt_sh.at[pl.ds(s * CNT_PER_TILE, CNT_PER_TILE)],
                       zsem)
      for k in range(CHUNK // 16):
        ones_v[pl.ds(16 * k, 16)] = jnp.ones((16,), _f32)

    def start_gather(j, buf):
      pltpu.async_copy(x_hbm.at[sidx.at[j]], buf, gsem)

    def wait_gather(j, buf):
      pltpu.make_async_copy(x_hbm.at[sidx.at[j]], buf, gsem).wait()

    def start_scat(j, buf):
      pltpu.async_copy(buf, acc_sh.at[didx.at[j]], ssem, add=True)
      if with_counts:
        pltpu.async_copy(ones_v, cnt_sh.at[didx.at[j]], csem, add=True)

    def wait_scat(j, buf):
      pltpu.make_async_copy(buf, acc_sh.at[didx.at[j]], ssem).wait()
      if with_counts:
        pltpu.make_async_copy(ones_v, cnt_sh.at[didx.at[j]], csem).wait()

    for h in range(NCH // SEG):
      pltpu.sync_copy(src_hbm.at[pl.ds(w * NCH + h * SEG, SEG)], sidx)
      pltpu.sync_copy(dst_hbm.at[pl.ds(w * NCH + h * SEG, SEG)], didx)

      start_gather(0, rows0)
      wait_gather(0, rows0)
      start_gather(1, rows1)
      if h == 0:
        pltpu.make_async_copy(
            z_hbm.at[pl.ds(s * ROWS_PER_TILE, ROWS_PER_TILE)],
            acc_sh.at[pl.ds(s * ROWS_PER_TILE, ROWS_PER_TILE)],
            zsem).wait()
        if with_counts:
          pltpu.make_async_copy(
              zc_hbm.at[pl.ds(s * CNT_PER_TILE, CNT_PER_TILE)],
              cnt_sh.at[pl.ds(s * CNT_PER_TILE, CNT_PER_TILE)],
              zsem).wait()
        plsc.subcore_barrier()
      start_scat(0, rows0)

      def pair(t, carry):
        for b, (cur, oth) in ((0, (rows1, rows0)), (1, (rows0, rows1))):
          j = 2 * t + 1 + b
          wait_gather(j, cur)
          wait_scat(j - 1, oth)
          start_gather(j + 1, oth)
          start_scat(j, cur)
        return carry

      lax.fori_loop(0, (SEG - 2) // 2, pair, 0)

      j = SEG - 1
      wait_gather(j, rows1)
      wait_scat(j - 1, rows0)
      start_scat(j, rows1)
      wait_scat(j, rows1)
    plsc.subcore_barrier()

    pltpu.sync_copy(acc_sh.at[pl.ds(s * ROWS_PER_TILE, ROWS_PER_TILE)],
                    p_hbm.at[c].at[pl.ds(s * ROWS_PER_TILE, ROWS_PER_TILE)])
    if with_counts:
      pltpu.sync_copy(
          cnt_sh.at[pl.ds(s * CNT_PER_TILE, CNT_PER_TILE)],
          cnt_hbm.at[pl.ds(c * N_CNT + s * CNT_PER_TILE, CNT_PER_TILE)])

  return pl.kernel(
      body,
      out_type=tuple(out_type) if with_counts else out_type[0],
      mesh=plsc.VectorSubcoreMesh(core_axis_name="c", subcore_axis_name="s"),
      scratch_types=scratch,
  )


_sc_agg_counts = _make_sc_agg(True)
_sc_agg = _make_sc_agg(False)

_GRID = 5
_R = N // _GRID


def _dense_body(p0, p1, c0, c1, x, wl, wr, b, o):
  cnt = jnp.maximum(c0[...] + c1[...], 1.0)
  mean = (p0[...] + p1[...]) / cnt
  h = (jnp.dot(mean, wl[...], preferred_element_type=_f32)
       + jnp.dot(x[...], wr[...], preferred_element_type=_f32) + b[...])
  o[...] = jnp.maximum(h, 0.0)


def _dense_final_body(p0, p1, c0, c1, x, wl, wr, b, wlin, blin, o):
  cnt = jnp.maximum(c0[...] + c1[...], 1.0)
  mean = (p0[...] + p1[...]) / cnt
  h = (jnp.dot(mean, wl[...], preferred_element_type=_f32)
       + jnp.dot(x[...], wr[...], preferred_element_type=_f32) + b[...])
  h = jnp.maximum(h, 0.0)
  o[...] = jnp.dot(h, wlin[...], preferred_element_type=_f32) + blin[...]


def _row_spec(d):
  return pl.BlockSpec((_R, d), lambda i: (i, 0))


def _full_spec(r, d):
  return pl.BlockSpec((r, d), lambda i: (0, 0))


_dense = pl.pallas_call(
    _dense_body,
    grid=(_GRID,),
    in_specs=[_row_spec(D), _row_spec(D), _row_spec(1), _row_spec(1),
              _row_spec(D), _full_spec(D, D), _full_spec(D, D),
              _full_spec(1, D)],
    out_specs=_row_spec(D),
    out_shape=jax.ShapeDtypeStruct((N, D), _f32),
)

_dense_final = pl.pallas_call(
    _dense_final_body,
    grid=(_GRID,),
    in_specs=[_row_spec(D), _row_spec(D), _row_spec(1), _row_spec(1),
              _row_spec(D), _full_spec(D, D), _full_spec(D, D),
              _full_spec(1, D), _full_spec(D, C), _full_spec(1, C)],
    out_specs=pl.BlockSpec((_R, C), lambda i: (i, 0)),
    out_shape=jax.ShapeDtypeStruct((N, C), _f32),
)


def kernel(x, edge_index, W1l, W1r, b1, W2l, W2r, b2, W3l, W3r, b3,
           Wlin, blin):
  src = edge_index[0].astype(jnp.int32)
  dst = edge_index[1].astype(jnp.int32)
  npad = E_PAD - E
  pad_iota = jnp.arange(npad, dtype=jnp.int32)
  src_p = jnp.concatenate([src, pad_iota % 64]).reshape(E_PAD // CHUNK, CHUNK)
  dst_p = jnp.concatenate([dst, N + (pad_iota % 16)]).reshape(
      E_PAD // CHUNK, CHUNK)

  zrows = jnp.zeros((N_ACC, D), _f32)
  zcnt = jnp.zeros((N_CNT,), _f32)

  p, cnt = _sc_agg_counts(x, src_p, dst_p, zrows, zcnt)
  cnt = cnt.reshape(NC, N_CNT)
  c0 = cnt[0, :N, None]
  c1 = cnt[1, :N, None]
  b1_ = b1.reshape(1, D)
  h = _dense(p[0, :N], p[1, :N], c0, c1, x, W1l, W1r, b1_)

  p = _sc_agg(h, src_p, dst_p, zrows)
  h = _dense(p[0, :N], p[1, :N], c0, c1, h, W2l, W2r, b2.reshape(1, D))

  p = _sc_agg(h, src_p, dst_p, zrows)
  out = _dense_final(p[0, :N], p[1, :N], c0, c1, h, W3l, W3r,
                     b3.reshape(1, D), Wlin, blin.reshape(1, C))
  return out


# speedup vs baseline: 11.1658x; 1.0253x over previous
import functools

import jax
import jax.numpy as jnp
from jax import lax
from jax.experimental import pallas as pl
from jax.experimental.pallas import tpu as pltpu
from jax.experimental.pallas import tpu_sc as plsc

N = 10000
E = 320000
D = 128
C = 64

NC = 2
NS = 16
NW = NC * NS

CHUNK = 128
NCH = 80
SEG = 40
EPW = NCH * CHUNK
E_PAD = EPW * NW
N_ACC = 10112
ROWS_PER_TILE = N_ACC // NS
N_CNT = 16384
CNT_PER_TILE = N_CNT // NS

_f32 = jnp.float32


def _make_sc_agg(with_counts: bool):
  out_type = [jax.ShapeDtypeStruct((NC, N_ACC, D), _f32)]
  if with_counts:
    out_type.append(jax.ShapeDtypeStruct((NC * N_CNT,), _f32))

  scratch = [
      pltpu.VMEM_SHARED((N_ACC, D), _f32),
      pltpu.VMEM((SEG, CHUNK), jnp.int32),
      pltpu.VMEM((SEG, CHUNK), jnp.int32),
      pltpu.VMEM((CHUNK, D), _f32),
      pltpu.VMEM((CHUNK, D), _f32),
      pltpu.SemaphoreType.DMA,
      pltpu.SemaphoreType.DMA,
      pltpu.SemaphoreType.DMA,
  ]
  if with_counts:
    scratch.insert(1, pltpu.VMEM_SHARED((N_CNT,), _f32))
    scratch.append(pltpu.VMEM((CHUNK,), _f32))
    scratch.append(pltpu.SemaphoreType.DMA)

  def body(*refs):
    if with_counts:
      (x_hbm, src_hbm, dst_hbm, z_hbm, zc_hbm, p_hbm, cnt_hbm,
       acc_sh, cnt_sh, sidx, didx, rows0, rows1, gsem, ssem, zsem,
       ones_v, csem) = refs
    else:
      (x_hbm, src_hbm, dst_hbm, z_hbm, p_hbm,
       acc_sh, sidx, didx, rows0, rows1, gsem, ssem, zsem) = refs

    c = lax.axis_index("c")
    s = lax.axis_index("s")
    w = c * NS + s

    pltpu.async_copy(z_hbm.at[pl.ds(s * ROWS_PER_TILE, ROWS_PER_TILE)],
                     acc_sh.at[pl.ds(s * ROWS_PER_TILE, ROWS_PER_TILE)],
                     zsem)
    if with_counts:
      pltpu.async_copy(zc_hbm.at[pl.ds(s * CNT_PER_TILE, CNT_PER_TILE)],
                       cnt_sh.at[pl.ds(s * CNT_PER_TILE, CNT_PER_TILE)],
                       zsem)
      for k in range(CHUNK // 16):
        ones_v[pl.ds(16 * k, 16)] = jnp.ones((16,), _f32)

    def start_gather(j, buf):
      pltpu.async_copy(x_hbm.at[sidx.at[j]], buf, gsem)

    def wait_gather(j, buf):
      pltpu.make_async_copy(x_hbm.at[sidx.at[j]], buf, gsem).wait()

    def start_scat(j, buf):
      pltpu.async_copy(buf, acc_sh.at[didx.at[j]], ssem, add=True)
      if with_counts:
        pltpu.async_copy(ones_v, cnt_sh.at[didx.at[j]], csem, add=True)

    def wait_scat(j, buf):
      pltpu.make_async_copy(buf, acc_sh.at[didx.at[j]], ssem).wait()
      if with_counts:
        pltpu.make_async_copy(ones_v, cnt_sh.at[didx.at[j]], csem).wait()

    for h in range(NCH // SEG):
      pltpu.sync_copy(src_hbm.at[pl.ds(w * NCH + h * SEG, SEG)], sidx)
      pltpu.sync_copy(dst_hbm.at[pl.ds(w * NCH + h * SEG, SEG)], didx)

      start_gather(0, rows0)
      wait_gather(0, rows0)
      start_gather(1, rows1)
      if h == 0:
        pltpu.make_async_copy(
            z_hbm.at[pl.ds(s * ROWS_PER_TILE, ROWS_PER_TILE)],
            acc_sh.at[pl.ds(s * ROWS_PER_TILE, ROWS_PER_TILE)],
            zsem).wait()
        if with_counts:
          pltpu.make_async_copy(
              zc_hbm.at[pl.ds(s * CNT_PER_TILE, CNT_PER_TILE)],
              cnt_sh.at[pl.ds(s * CNT_PER_TILE, CNT_PER_TILE)],
              zsem).wait()
        plsc.subcore_barrier()
      start_scat(0, rows0)

      def pair(t, carry):
        for b, (cur, oth) in ((0, (rows1, rows0)), (1, (rows0, rows1))):
          j = 2 * t + 1 + b
          wait_gather(j, cur)
          wait_scat(j - 1, oth)
          start_gather(j + 1, oth)
          start_scat(j, cur)
        return carry

      lax.fori_loop(0, (SEG - 2) // 2, pair, 0)

      j = SEG - 1
      wait_gather(j, rows1)
      wait_scat(j - 1, rows0)
      start_scat(j, rows1)
      wait_scat(j, rows1)
    plsc.subcore_barrier()

    pltpu.sync_copy(acc_sh.at[pl.ds(s * ROWS_PER_TILE, ROWS_PER_TILE)],
                    p_hbm.at[c].at[pl.ds(s * ROWS_PER_TILE, ROWS_PER_TILE)])
    if with_counts:
      pltpu.sync_copy(
          cnt_sh.at[pl.ds(s * CNT_PER_TILE, CNT_PER_TILE)],
          cnt_hbm.at[pl.ds(c * N_CNT + s * CNT_PER_TILE, CNT_PER_TILE)])

  return pl.kernel(
      body,
      out_type=tuple(out_type) if with_counts else out_type[0],
      mesh=plsc.VectorSubcoreMesh(core_axis_name="c", subcore_axis_name="s"),
      scratch_types=scratch,
  )


_sc_agg_counts = _make_sc_agg(True)
_sc_agg = _make_sc_agg(False)

_GRID = 5
_R = N // _GRID


def _dense_body(p0, p1, c0, c1, x, wl, wr, b, o):
  cnt = jnp.maximum(c0[0] + c1[0], 1.0)
  mean = (p0[0] + p1[0]) / cnt
  h = (jnp.dot(mean, wl[...], preferred_element_type=_f32)
       + jnp.dot(x[...], wr[...], preferred_element_type=_f32) + b[...])
  o[...] = jnp.maximum(h, 0.0)


def _dense_final_body(p0, p1, c0, c1, x, wl, wr, b, wlin, blin, o):
  cnt = jnp.maximum(c0[0] + c1[0], 1.0)
  mean = (p0[0] + p1[0]) / cnt
  h = (jnp.dot(mean, wl[...], preferred_element_type=_f32)
       + jnp.dot(x[...], wr[...], preferred_element_type=_f32) + b[...])
  h = jnp.maximum(h, 0.0)
  o[...] = jnp.dot(h, wlin[...], preferred_element_type=_f32) + blin[...]


def _core_spec(core, d):
  return pl.BlockSpec((1, _R, d), lambda i, _c=core: (_c, i, 0))


def _row_spec(d):
  return pl.BlockSpec((_R, d), lambda i: (i, 0))


def _full_spec(r, d):
  return pl.BlockSpec((r, d), lambda i: (0, 0))


def _dense(p, cnt, x, wl, wr, b):
  return pl.pallas_call(
      _dense_body,
      grid=(_GRID,),
      in_specs=[_core_spec(0, D), _core_spec(1, D), _core_spec(0, 1),
                _core_spec(1, 1), _row_spec(D), _full_spec(D, D),
                _full_spec(D, D), _full_spec(1, D)],
      out_specs=_row_spec(D),
      out_shape=jax.ShapeDtypeStruct((N, D), _f32),
  )(p, p, cnt, cnt, x, wl, wr, b)


def _dense_final(p, cnt, x, wl, wr, b, wlin, blin):
  return pl.pallas_call(
      _dense_final_body,
      grid=(_GRID,),
      in_specs=[_core_spec(0, D), _core_spec(1, D), _core_spec(0, 1),
                _core_spec(1, 1), _row_spec(D), _full_spec(D, D),
                _full_spec(D, D), _full_spec(1, D), _full_spec(D, C),
                _full_spec(1, C)],
      out_specs=pl.BlockSpec((_R, C), lambda i: (i, 0)),
      out_shape=jax.ShapeDtypeStruct((N, C), _f32),
  )(p, p, cnt, cnt, x, wl, wr, b, wlin, blin)


def kernel(x, edge_index, W1l, W1r, b1, W2l, W2r, b2, W3l, W3r, b3,
           Wlin, blin):
  src = edge_index[0].astype(jnp.int32)
  dst = edge_index[1].astype(jnp.int32)
  npad = E_PAD - E
  pad_iota = jnp.arange(npad, dtype=jnp.int32)
  src_p = jnp.concatenate([src, pad_iota % 64]).reshape(E_PAD // CHUNK, CHUNK)
  dst_p = jnp.concatenate([dst, N + (pad_iota % 16)]).reshape(
      E_PAD // CHUNK, CHUNK)

  zrows = jnp.zeros((N_ACC, D), _f32)
  zcnt = jnp.zeros((N_CNT,), _f32)

  p, cnt = _sc_agg_counts(x, src_p, dst_p, zrows, zcnt)
  cnt = cnt.reshape(NC, N_CNT, 1)
  h = _dense(p, cnt, x, W1l, W1r, b1.reshape(1, D))

  p = _sc_agg(h, src_p, dst_p, zrows)
  h = _dense(p, cnt, h, W2l, W2r, b2.reshape(1, D))

  p = _sc_agg(h, src_p, dst_p, zrows)
  out = _dense_final(p, cnt, h, W3l, W3r, b3.reshape(1, D), Wlin,
                     blin.reshape(1, C))
  return out


# speedup vs baseline: 11.4798x; 1.0281x over previous
import functools

import jax
import jax.numpy as jnp
from jax import lax
from jax.experimental import pallas as pl
from jax.experimental.pallas import tpu as pltpu
from jax.experimental.pallas import tpu_sc as plsc

N = 10000
E = 320000
D = 128
C = 64

NC = 2
NS = 16
NW = NC * NS

CHUNK = 128
NCH = 80
SEG = 40
EPW = NCH * CHUNK
E_PAD = EPW * NW
N_ACC = 10112
ROWS_PER_TILE = N_ACC // NS
N_CNT = 10240
CNT_PER_TILE = N_CNT // NS
CNT_ROWS = N_CNT // 128

_f32 = jnp.float32


def _make_sc_agg(with_counts: bool):
  out_type = [jax.ShapeDtypeStruct((NC, N_ACC, D), _f32)]
  if with_counts:
    out_type.append(jax.ShapeDtypeStruct((NC * N_CNT,), _f32))

  scratch = [
      pltpu.VMEM_SHARED((N_ACC, D), _f32),
      pltpu.VMEM((SEG, CHUNK), jnp.int32),
      pltpu.VMEM((SEG, CHUNK), jnp.int32),
      pltpu.VMEM((CHUNK, D), _f32),
      pltpu.VMEM((CHUNK, D), _f32),
      pltpu.SemaphoreType.DMA,
      pltpu.SemaphoreType.DMA,
      pltpu.SemaphoreType.DMA,
  ]
  if with_counts:
    scratch.insert(1, pltpu.VMEM_SHARED((N_CNT,), _f32))
    scratch.append(pltpu.VMEM((CHUNK,), _f32))
    scratch.append(pltpu.SemaphoreType.DMA)

  def body(*refs):
    if with_counts:
      (x_hbm, src_hbm, dst_hbm, z_hbm, zc_hbm, p_hbm, cnt_hbm,
       acc_sh, cnt_sh, sidx, didx, rows0, rows1, gsem, ssem, zsem,
       ones_v, csem) = refs
    else:
      (x_hbm, src_hbm, dst_hbm, z_hbm, p_hbm,
       acc_sh, sidx, didx, rows0, rows1, gsem, ssem, zsem) = refs

    c = lax.axis_index("c")
    s = lax.axis_index("s")
    w = c * NS + s

    pltpu.async_copy(z_hbm.at[pl.ds(s * ROWS_PER_TILE, ROWS_PER_TILE)],
                     acc_sh.at[pl.ds(s * ROWS_PER_TILE, ROWS_PER_TILE)],
                     zsem)
    if with_counts:
      pltpu.async_copy(zc_hbm.at[pl.ds(s * CNT_PER_TILE, CNT_PER_TILE)],
                       cnt_sh.at[pl.ds(s * CNT_PER_TILE, CNT_PER_TILE)],
                       zsem)
      for k in range(CHUNK // 16):
        ones_v[pl.ds(16 * k, 16)] = jnp.ones((16,), _f32)

    def start_gather(j, buf):
      pltpu.async_copy(x_hbm.at[sidx.at[j]], buf, gsem)

    def wait_gather(j, buf):
      pltpu.make_async_copy(x_hbm.at[sidx.at[j]], buf, gsem).wait()

    def start_scat(j, buf):
      pltpu.async_copy(buf, acc_sh.at[didx.at[j]], ssem, add=True)
      if with_counts:
        pltpu.async_copy(ones_v, cnt_sh.at[didx.at[j]], csem, add=True)

    def wait_scat(j, buf):
      pltpu.make_async_copy(buf, acc_sh.at[didx.at[j]], ssem).wait()
      if with_counts:
        pltpu.make_async_copy(ones_v, cnt_sh.at[didx.at[j]], csem).wait()

    for h in range(NCH // SEG):
      pltpu.sync_copy(src_hbm.at[pl.ds(w * NCH + h * SEG, SEG)], sidx)
      pltpu.sync_copy(dst_hbm.at[pl.ds(w * NCH + h * SEG, SEG)], didx)

      start_gather(0, rows0)
      wait_gather(0, rows0)
      start_gather(1, rows1)
      if h == 0:
        pltpu.make_async_copy(
            z_hbm.at[pl.ds(s * ROWS_PER_TILE, ROWS_PER_TILE)],
            acc_sh.at[pl.ds(s * ROWS_PER_TILE, ROWS_PER_TILE)],
            zsem).wait()
        if with_counts:
          pltpu.make_async_copy(
              zc_hbm.at[pl.ds(s * CNT_PER_TILE, CNT_PER_TILE)],
              cnt_sh.at[pl.ds(s * CNT_PER_TILE, CNT_PER_TILE)],
              zsem).wait()
        plsc.subcore_barrier()
      start_scat(0, rows0)

      def pair(t, carry):
        for b, (cur, oth) in ((0, (rows1, rows0)), (1, (rows0, rows1))):
          j = 2 * t + 1 + b
          wait_gather(j, cur)
          wait_scat(j - 1, oth)
          start_gather(j + 1, oth)
          start_scat(j, cur)
        return carry

      lax.fori_loop(0, (SEG - 2) // 2, pair, 0)

      j = SEG - 1
      wait_gather(j, rows1)
      wait_scat(j - 1, rows0)
      start_scat(j, rows1)
      wait_scat(j, rows1)
    plsc.subcore_barrier()

    pltpu.sync_copy(acc_sh.at[pl.ds(s * ROWS_PER_TILE, ROWS_PER_TILE)],
                    p_hbm.at[c].at[pl.ds(s * ROWS_PER_TILE, ROWS_PER_TILE)])
    if with_counts:
      pltpu.sync_copy(
          cnt_sh.at[pl.ds(s * CNT_PER_TILE, CNT_PER_TILE)],
          cnt_hbm.at[pl.ds(c * N_CNT + s * CNT_PER_TILE, CNT_PER_TILE)])

  return pl.kernel(
      body,
      out_type=tuple(out_type) if with_counts else out_type[0],
      mesh=plsc.VectorSubcoreMesh(core_axis_name="c", subcore_axis_name="s"),
      scratch_types=scratch,
  )


_sc_agg_counts = _make_sc_agg(True)
_sc_agg = _make_sc_agg(False)

_GRID = 5
_R = N // _GRID


def _inv_cnt_col(c0, c1):
  i = pl.program_id(0)
  r = 1.0 / jnp.maximum(c0[0] + c1[0], 1.0)
  g_k = lax.broadcasted_iota(jnp.int32, (_R, CNT_ROWS), 0) + _R * i
  k_j = lax.broadcasted_iota(jnp.int32, (_R, CNT_ROWS), 1)
  sel_row = (k_j == g_k // 128).astype(_f32)
  b_full = jnp.dot(sel_row, r, preferred_element_type=_f32,
                   precision=lax.Precision.HIGHEST)
  g_l = lax.broadcasted_iota(jnp.int32, (_R, 128), 0) + _R * i
  lane = lax.broadcasted_iota(jnp.int32, (_R, 128), 1)
  picked = jnp.where(lane == g_l % 128, b_full, 0.0)
  return jnp.sum(picked, axis=1, keepdims=True)


def _dense_body(p0, p1, c0, c1, x, wl, wr, b, o):
  mean = (p0[0] + p1[0]) * _inv_cnt_col(c0, c1)
  h = (jnp.dot(mean, wl[...], preferred_element_type=_f32)
       + jnp.dot(x[...], wr[...], preferred_element_type=_f32) + b[...])
  o[...] = jnp.maximum(h, 0.0)


def _dense_final_body(p0, p1, c0, c1, x, wl, wr, b, wlin, blin, o):
  mean = (p0[0] + p1[0]) * _inv_cnt_col(c0, c1)
  h = (jnp.dot(mean, wl[...], preferred_element_type=_f32)
       + jnp.dot(x[...], wr[...], preferred_element_type=_f32) + b[...])
  h = jnp.maximum(h, 0.0)
  o[...] = jnp.dot(h, wlin[...], preferred_element_type=_f32) + blin[...]


def _core_spec(core, d):
  return pl.BlockSpec((1, _R, d), lambda i, _c=core: (_c, i, 0))


def _cnt_spec(core):
  return pl.BlockSpec((1, CNT_ROWS, 128), lambda i, _c=core: (_c, 0, 0))


def _row_spec(d):
  return pl.BlockSpec((_R, d), lambda i: (i, 0))


def _full_spec(r, d):
  return pl.BlockSpec((r, d), lambda i: (0, 0))


def _dense(p, cnt, x, wl, wr, b):
  return pl.pallas_call(
      _dense_body,
      grid=(_GRID,),
      in_specs=[_core_spec(0, D), _core_spec(1, D), _cnt_spec(0),
                _cnt_spec(1), _row_spec(D), _full_spec(D, D),
                _full_spec(D, D), _full_spec(1, D)],
      out_specs=_row_spec(D),
      out_shape=jax.ShapeDtypeStruct((N, D), _f32),
  )(p, p, cnt, cnt, x, wl, wr, b)


def _dense_final(p, cnt, x, wl, wr, b, wlin, blin):
  return pl.pallas_call(
      _dense_final_body,
      grid=(_GRID,),
      in_specs=[_core_spec(0, D), _core_spec(1, D), _cnt_spec(0),
                _cnt_spec(1), _row_spec(D), _full_spec(D, D),
                _full_spec(D, D), _full_spec(1, D), _full_spec(D, C),
                _full_spec(1, C)],
      out_specs=pl.BlockSpec((_R, C), lambda i: (i, 0)),
      out_shape=jax.ShapeDtypeStruct((N, C), _f32),
  )(p, p, cnt, cnt, x, wl, wr, b, wlin, blin)


def kernel(x, edge_index, W1l, W1r, b1, W2l, W2r, b2, W3l, W3r, b3,
           Wlin, blin):
  src = edge_index[0].astype(jnp.int32)
  dst = edge_index[1].astype(jnp.int32)
  npad = E_PAD - E
  pad_iota = jnp.arange(npad, dtype=jnp.int32)
  src_p = jnp.concatenate([src, pad_iota % 64]).reshape(E_PAD // CHUNK, CHUNK)
  dst_p = jnp.concatenate([dst, N + (pad_iota % 16)]).reshape(
      E_PAD // CHUNK, CHUNK)

  zrows = jnp.zeros((N_ACC, D), _f32)
  zcnt = jnp.zeros((N_CNT,), _f32)

  p, cnt = _sc_agg_counts(x, src_p, dst_p, zrows, zcnt)
  cnt = cnt.reshape(NC, CNT_ROWS, 128)
  h = _dense(p, cnt, x, W1l, W1r, b1.reshape(1, D))

  p = _sc_agg(h, src_p, dst_p, zrows)
  h = _dense(p, cnt, h, W2l, W2r, b2.reshape(1, D))

  p = _sc_agg(h, src_p, dst_p, zrows)
  out = _dense_final(p, cnt, h, W3l, W3r, b3.reshape(1, D), Wlin,
                     blin.reshape(1, C))
  return out
